# baseline (device time: 24707 ns/iter reference)
import jax
import jax.numpy as jnp
from jax import lax
from jax.experimental import pallas as pl
from jax.experimental.pallas import tpu as pltpu

N_GLOBAL = 2048
EPS = 1e-5
K = 8


def kernel(x, gamma, beta):
    m, n = x.shape
    bm = m // K
    gamma2 = gamma.reshape(1, n)
    beta2 = beta.reshape(1, n)

    def body(
        x_hbm, g_ref, b_ref, o_hbm,
        xv, ov, stats, send_buf, recv,
        in_sems, out_sems, send_sem, recv_sem,
    ):
        my_x = lax.axis_index("x")
        my_y = lax.axis_index("y")
        peer = (my_x, 1 - my_y)

        copies_in = []
        for k in range(K):
            cp = pltpu.make_async_copy(
                x_hbm.at[pl.ds(k * bm, bm), :],
                xv.at[pl.ds(k * bm, bm), :],
                in_sems.at[k],
            )
            cp.start()
            copies_in.append(cp)

        for k in range(K):
            copies_in[k].wait()
            xb = xv[pl.ds(k * bm, bm), :]
            stats[pl.ds(k * bm, bm), 0:1] = jnp.sum(xb, axis=1, keepdims=True)
            stats[pl.ds(k * bm, bm), 1:2] = jnp.sum(xb * xb, axis=1, keepdims=True)

        barrier = pltpu.get_barrier_semaphore()
        pl.semaphore_signal(
            barrier, inc=1, device_id=peer, device_id_type=pl.DeviceIdType.MESH
        )
        pl.semaphore_wait(barrier, 1)

        send_buf[...] = jnp.transpose(stats[...], (1, 0))
        rdma = pltpu.make_async_remote_copy(
            src_ref=send_buf,
            dst_ref=recv,
            send_sem=send_sem,
            recv_sem=recv_sem,
            device_id=peer,
            device_id_type=pl.DeviceIdType.MESH,
        )
        rdma.start()
        rdma.wait()

        rt = jnp.transpose(recv[...], (1, 0))
        tot1 = stats[:, 0:1] + rt[:, 0:1]
        tot2 = stats[:, 1:2] + rt[:, 1:2]
        mean = tot1 / N_GLOBAL
        var = tot2 / N_GLOBAL - mean * mean
        stats[:, 0:1] = mean
        stats[:, 1:2] = lax.rsqrt(var + EPS)

        copies_out = []
        for k in range(K):
            sl = pl.ds(k * bm, bm)
            mu = stats[sl, 0:1]
            rstd = stats[sl, 1:2]
            ov[sl, :] = (
                (xv[sl, :] - mu) * rstd * g_ref[...] + b_ref[...]
            ).astype(jnp.bfloat16)
            cp = pltpu.make_async_copy(ov.at[sl, :], o_hbm.at[sl, :], out_sems.at[k])
            cp.start()
            copies_out.append(cp)
        for cp in copies_out:
            cp.wait()

    return pl.pallas_call(
        body,
        out_shape=jax.ShapeDtypeStruct((m, n), jnp.bfloat16),
        in_specs=[
            pl.BlockSpec(memory_space=pl.ANY),
            pl.BlockSpec(memory_space=pltpu.VMEM),
            pl.BlockSpec(memory_space=pltpu.VMEM),
        ],
        out_specs=pl.BlockSpec(memory_space=pl.ANY),
        scratch_shapes=[
            pltpu.VMEM((m, n), jnp.float32),
            pltpu.VMEM((m, n), jnp.bfloat16),
            pltpu.VMEM((m, 2), jnp.float32),
            pltpu.VMEM((2, m), jnp.float32),
            pltpu.VMEM((2, m), jnp.float32),
            pltpu.SemaphoreType.DMA((K,)),
            pltpu.SemaphoreType.DMA((K,)),
            pltpu.SemaphoreType.DMA,
            pltpu.SemaphoreType.DMA,
        ],
        compiler_params=pltpu.CompilerParams(
            collective_id=0, vmem_limit_bytes=64 * 1024 * 1024
        ),
    )(x, gamma2, beta2)
